# baseline (speedup 1.0000x reference)
import functools

import jax
import jax.numpy as jnp
from jax.experimental import pallas as pl
from jax.experimental.pallas import tpu as pltpu


def _fused_kernel2(adj_a, adj_b, x_ref, w1_ref, b1_ref, w2_ref, b2_ref, o_ref,
                   y1_scr, y2_scr, *, bm):
    l = pl.program_id(0)
    i = pl.program_id(1)

    @pl.when((l == 0) & (i == 0))
    def _():
        y1_scr[...] = jnp.dot(
            x_ref[...].astype(jnp.bfloat16),
            w1_ref[...].astype(jnp.bfloat16),
            preferred_element_type=jnp.float32,
        ).astype(jnp.bfloat16)

    @pl.when(l == 0)
    def _():
        for half, ref in ((0, adj_a), (1, adj_b)):
            a = jnp.dot(
                ref[0].astype(jnp.bfloat16), y1_scr[...],
                preferred_element_type=jnp.float32,
            )
            h = jnp.maximum(a + b1_ref[...], 0.0)
            base = (2 * i + half) * bm
            y2_scr[pl.ds(base, bm), :] = jnp.dot(
                h.astype(jnp.bfloat16), w2_ref[...],
                preferred_element_type=jnp.float32,
            ).astype(jnp.bfloat16)

    @pl.when(l == 1)
    def _():
        for half, ref in ((0, adj_a), (1, adj_b)):
            a = jnp.dot(
                ref[0].astype(jnp.bfloat16), y2_scr[...],
                preferred_element_type=jnp.float32,
            )
            o_ref[pl.ds(half * bm, bm), :] = a + b2_ref[...]


def _gcn2(x, adj, W1, b1, W2, b2, *, bm, interpret=False):
    N, F_in = x.shape
    H = W1.shape[1]
    C = W2.shape[1]
    assert N % (2 * bm) == 0
    grid = (2, N // (2 * bm))

    return pl.pallas_call(
        functools.partial(_fused_kernel2, bm=bm),
        grid=grid,
        in_specs=[
            pl.BlockSpec((1, bm, N), lambda l, i: (l, 2 * i, 0)),
            pl.BlockSpec((1, bm, N), lambda l, i: (l, 2 * i + 1, 0)),
            pl.BlockSpec((N, F_in), lambda l, i: (0, 0)),
            pl.BlockSpec((F_in, H), lambda l, i: (0, 0)),
            pl.BlockSpec((1, H), lambda l, i: (0, 0)),
            pl.BlockSpec((H, C), lambda l, i: (0, 0)),
            pl.BlockSpec((1, C), lambda l, i: (0, 0)),
        ],
        out_specs=pl.BlockSpec((2 * bm, C), lambda l, i: (l * i, 0)),
        out_shape=jax.ShapeDtypeStruct((N, C), jnp.float32),
        scratch_shapes=[
            pltpu.VMEM((N, H), jnp.bfloat16),
            pltpu.VMEM((N, C), jnp.bfloat16),
        ],
        compiler_params=pltpu.CompilerParams(
            dimension_semantics=("arbitrary", "arbitrary"),
        ),
        interpret=interpret,
    )(adj, adj, x, W1, b1.reshape(1, H), W2, b2.reshape(1, C))


def kernel(x, adj, W1, b1, W2, b2):
    return _gcn2(x, adj, W1, b1, W2, b2, bm=200)
